# Initial kernel scaffold; baseline (speedup 1.0000x reference)
import jax
import jax.numpy as jnp
from jax.experimental import pallas as pl


def kernel(w0, w1, w2, r1, r2):
    raise NotImplementedError("write your pallas kernel here")


# speedup vs baseline: 102.9554x; 102.9554x over previous
import jax
import jax.numpy as jnp
from jax import lax
from jax.experimental import pallas as pl
from jax.experimental.pallas import tpu as pltpu
from jax.experimental.pallas import tpu_sc as plsc

NC = 2
NS = 16
L = 16


def _build(E, N, CH, interpret=False):
  SLICE = L * (-(-N // (NS * L)))
  NPAD = NS * SLICE
  ET = E // NS
  NCH = ET // CH
  assert ET % CH == 0 and CH % L == 0

  mesh = plsc.VectorSubcoreMesh(core_axis_name="c", subcore_axis_name="s",
                                num_cores=NC, num_subcores=NS)

  def body(w0_hbm, w1_hbm, w2_hbm, r1_hbm, r2_hbm,
           lfw0_hbm, lfw1_hbm, lfw2_hbm,
           table_v, idx_v, w_v, acc_v, tmp_v,
           rot_sh, inv_sh):
    c = lax.axis_index("c")
    s = lax.axis_index("s")

    def zero_body(i, _):
      table_v[pl.ds(i * L, L)] = jnp.zeros((L,), jnp.float32)
      return _
    lax.fori_loop(0, NPAD // L, zero_body, None)

    def chunk_scatter(k, _):
      base = s * ET + k * CH

      @pl.when(c == 0)
      def _u():
        pltpu.sync_copy(r1_hbm.at[pl.ds(base, CH)], idx_v)
        pltpu.sync_copy(w1_hbm.at[pl.ds(base, CH)], w_v)

      @pl.when(c == 1)
      def _i():
        pltpu.sync_copy(r2_hbm.at[pl.ds(base, CH)], idx_v)
        pltpu.sync_copy(w2_hbm.at[pl.ds(base, CH)], w_v)

      def vec_body(i, _):
        sl = pl.ds(i * L, L)
        plsc.addupdate_scatter(table_v, [idx_v[sl]], jnp.exp(w_v[sl]))
        return _
      lax.fori_loop(0, CH // L, vec_body, None)
      return _
    lax.fori_loop(0, NCH, chunk_scatter, None)

    o = s * SLICE
    pltpu.sync_copy(w0_hbm.at[pl.ds(c * NPAD + o, SLICE)], acc_v)

    def exp_body(i, _):
      sl = pl.ds(i * L, L)
      acc_v[sl] = jnp.exp(acc_v[sl])
      return _
    lax.fori_loop(0, SLICE // L, exp_body, None)

    for r in range(NS):
      src = ((s + r) % NS) * SLICE
      pltpu.sync_copy(table_v.at[pl.ds(src, SLICE)],
                      rot_sh.at[pl.ds(o, SLICE)])
      plsc.subcore_barrier()
      p = ((s + NS - r) % NS) * SLICE
      pltpu.sync_copy(rot_sh.at[pl.ds(p, SLICE)], tmp_v)

      def add_body(i, _):
        sl = pl.ds(i * L, L)
        acc_v[sl] = acc_v[sl] + tmp_v[sl]
        return _
      lax.fori_loop(0, SLICE // L, add_body, None)
      plsc.subcore_barrier()

    def inv_body(i, _):
      sl = pl.ds(i * L, L)
      acc_v[sl] = 1.0 / acc_v[sl]
      return _
    lax.fori_loop(0, SLICE // L, inv_body, None)

    pltpu.sync_copy(w0_hbm.at[pl.ds(c * NPAD + o, SLICE)], tmp_v)

    def lfw0_body(i, _):
      sl = pl.ds(i * L, L)
      tmp_v[sl] = jnp.exp(tmp_v[sl]) * acc_v[sl]
      return _
    lax.fori_loop(0, SLICE // L, lfw0_body, None)

    pltpu.sync_copy(tmp_v, lfw0_hbm.at[pl.ds(c * NPAD + o, SLICE)])
    pltpu.sync_copy(acc_v, inv_sh.at[pl.ds(o, SLICE)])
    plsc.subcore_barrier()

    pltpu.sync_copy(inv_sh, table_v)

    def chunk_norm(k, _):
      base = s * ET + k * CH

      @pl.when(c == 0)
      def _u():
        pltpu.sync_copy(r1_hbm.at[pl.ds(base, CH)], idx_v)
        pltpu.sync_copy(w1_hbm.at[pl.ds(base, CH)], w_v)

      @pl.when(c == 1)
      def _i():
        pltpu.sync_copy(r2_hbm.at[pl.ds(base, CH)], idx_v)
        pltpu.sync_copy(w2_hbm.at[pl.ds(base, CH)], w_v)

      def vec_body(i, _):
        sl = pl.ds(i * L, L)
        g = plsc.load_gather(table_v, [idx_v[sl]])
        w_v[sl] = jnp.exp(w_v[sl]) * g
        return _
      lax.fori_loop(0, CH // L, vec_body, None)

      @pl.when(c == 0)
      def _ou():
        pltpu.sync_copy(w_v, lfw1_hbm.at[pl.ds(base, CH)])

      @pl.when(c == 1)
      def _oi():
        pltpu.sync_copy(w_v, lfw2_hbm.at[pl.ds(base, CH)])
      return _
    lax.fori_loop(0, NCH, chunk_norm, None)

  kern = pl.kernel(
      body,
      out_type=(
          jax.ShapeDtypeStruct((NC * NPAD,), jnp.float32),
          jax.ShapeDtypeStruct((E,), jnp.float32),
          jax.ShapeDtypeStruct((E,), jnp.float32),
      ),
      mesh=mesh,
      scratch_types=[
          pltpu.VMEM((NPAD,), jnp.float32),
          pltpu.VMEM((CH,), jnp.int32),
          pltpu.VMEM((CH,), jnp.float32),
          pltpu.VMEM((SLICE,), jnp.float32),
          pltpu.VMEM((SLICE,), jnp.float32),
          pltpu.VMEM_SHARED((NS * SLICE,), jnp.float32),
          pltpu.VMEM_SHARED((NPAD,), jnp.float32),
      ],
      compiler_params=pltpu.CompilerParams(needs_layout_passes=False),
      interpret=interpret,
  )
  return kern, NPAD


def kernel(w0, w1, w2, r1, r2):
  E = int(w1.shape[0])
  N = int(w0.shape[0])
  kern, NPAD = _build(E, N, CH=2000)
  w0p = jnp.full((NC * NPAD,), -88.0, jnp.float32).at[:N].set(w0)
  lfw0f, lfw1, lfw2 = kern(w0p, w1, w2,
                           r1.astype(jnp.int32), r2.astype(jnp.int32))
  return lfw0f[:N], lfw1, lfw2


# speedup vs baseline: 190.9574x; 1.8548x over previous
import jax
import jax.numpy as jnp
from jax import lax
from jax.experimental import pallas as pl
from jax.experimental.pallas import tpu as pltpu
from jax.experimental.pallas import tpu_sc as plsc

NC = 2
NS = 16
L = 16


def _build(E, N, CH, interpret=False):
  SLICE = 2 * L * (-(-N // (NS * 2 * L)))
  HSLICE = SLICE // 2
  NPAD = NS * SLICE
  ET = E // NS
  NCH = ET // CH
  NPAIR = NCH // 2
  assert ET % CH == 0 and CH % L == 0 and NCH % 2 == 0

  mesh = plsc.VectorSubcoreMesh(core_axis_name="c", subcore_axis_name="s",
                                num_cores=NC, num_subcores=NS)

  def body(w0_hbm, w1_hbm, w2_hbm, r1_hbm, r2_hbm,
           lfw0_hbm, lfw1_hbm, lfw2_hbm,
           table_v, idx_v, w_v, o_v, acc_v, tmp_v,
           rot_sh, inv_sh,
           sem_zi, sem_i0, sem_i1, sem_w0, sem_w1, sem_o0, sem_o1):
    c = lax.axis_index("c")
    s = lax.axis_index("s")
    sem_i = (sem_i0, sem_i1)
    sem_w = (sem_w0, sem_w1)
    sem_o = (sem_o0, sem_o1)

    def start_in(g, b):
      base = s * ET + g * CH
      dst_i = idx_v.at[pl.ds(b * CH, CH)]
      dst_w = w_v.at[pl.ds(b * CH, CH)]

      @pl.when(c == 0)
      def _u():
        pltpu.async_copy(r1_hbm.at[pl.ds(base, CH)], dst_i, sem_i[b])
        pltpu.async_copy(w1_hbm.at[pl.ds(base, CH)], dst_w, sem_w[b])

      @pl.when(c == 1)
      def _i():
        pltpu.async_copy(r2_hbm.at[pl.ds(base, CH)], dst_i, sem_i[b])
        pltpu.async_copy(w2_hbm.at[pl.ds(base, CH)], dst_w, sem_w[b])

    def wait_in(b):
      pltpu.make_async_copy(r1_hbm.at[pl.ds(0, CH)],
                            idx_v.at[pl.ds(b * CH, CH)], sem_i[b]).wait()
      pltpu.make_async_copy(w1_hbm.at[pl.ds(0, CH)],
                            w_v.at[pl.ds(b * CH, CH)], sem_w[b]).wait()

    def start_out(g, b):
      base = s * ET + g * CH
      src = o_v.at[pl.ds(b * CH, CH)]

      @pl.when(c == 0)
      def _u():
        pltpu.async_copy(src, lfw1_hbm.at[pl.ds(base, CH)], sem_o[b])

      @pl.when(c == 1)
      def _i():
        pltpu.async_copy(src, lfw2_hbm.at[pl.ds(base, CH)], sem_o[b])

    def wait_out(b):
      pltpu.make_async_copy(o_v.at[pl.ds(b * CH, CH)],
                            lfw1_hbm.at[pl.ds(0, CH)], sem_o[b]).wait()

    ztab = pltpu.make_async_copy(w0_hbm.at[pl.ds(NC * NPAD, NPAD)],
                                 table_v, sem_zi)
    ztab.start()
    start_in(0, 0)
    start_in(1, 1)
    ztab.wait()

    def scatter_chunk(g, b):
      def vec_body(i, _):
        sl = pl.ds(b * CH + i * L, L)
        plsc.addupdate_scatter(table_v, [idx_v[sl]], jnp.exp(w_v[sl]))
        return _
      lax.fori_loop(0, CH // L, vec_body, None, unroll=4)

    def pair_scatter(gp, _):
      g0 = 2 * gp
      wait_in(0)
      scatter_chunk(g0, 0)

      @pl.when(gp + 1 < NPAIR)
      def _n0():
        start_in(g0 + 2, 0)
      wait_in(1)
      scatter_chunk(g0 + 1, 1)

      @pl.when(gp + 1 < NPAIR)
      def _n1():
        start_in(g0 + 3, 1)
      return _
    lax.fori_loop(0, NPAIR, pair_scatter, None)

    o = s * SLICE
    pltpu.sync_copy(w0_hbm.at[pl.ds(c * NPAD + o, SLICE)], acc_v)

    def exp_body(i, _):
      sl = pl.ds(i * L, L)
      acc_v[sl] = jnp.exp(acc_v[sl])
      return _
    lax.fori_loop(0, SLICE // L, exp_body, None, unroll=4)

    for r in range(NS):
      for h in range(2):
        src = ((s + r) % NS) * SLICE + h * HSLICE
        pltpu.sync_copy(table_v.at[pl.ds(src, HSLICE)],
                        rot_sh.at[pl.ds(s * HSLICE, HSLICE)])
        plsc.subcore_barrier()
        p = ((s + NS - r) % NS) * HSLICE
        pltpu.sync_copy(rot_sh.at[pl.ds(p, HSLICE)], tmp_v)

        def add_body(i, _):
          sl = pl.ds(i * L, L)
          hl = pl.ds(h * HSLICE + i * L, L)
          acc_v[hl] = acc_v[hl] + tmp_v[sl]
          return _
        lax.fori_loop(0, HSLICE // L, add_body, None, unroll=4)
        plsc.subcore_barrier()

    def inv_body(i, _):
      sl = pl.ds(i * L, L)
      acc_v[sl] = 1.0 / acc_v[sl]
      return _
    lax.fori_loop(0, SLICE // L, inv_body, None, unroll=4)

    for h in range(2):
      pltpu.sync_copy(w0_hbm.at[pl.ds(c * NPAD + o + h * HSLICE, HSLICE)],
                      tmp_v)

      def lfw0_body(i, _):
        sl = pl.ds(i * L, L)
        tmp_v[sl] = jnp.exp(tmp_v[sl]) * acc_v[pl.ds(h * HSLICE + i * L, L)]
        return _
      lax.fori_loop(0, HSLICE // L, lfw0_body, None, unroll=4)
      pltpu.sync_copy(tmp_v,
                      lfw0_hbm.at[pl.ds(c * NPAD + o + h * HSLICE, HSLICE)])

    pltpu.sync_copy(acc_v, inv_sh.at[pl.ds(o, SLICE)])
    plsc.subcore_barrier()

    pltpu.sync_copy(inv_sh, table_v)
    start_in(0, 0)
    start_in(1, 1)

    def norm_chunk(g, b):
      def vec_body(i, _):
        sl = pl.ds(b * CH + i * L, L)
        g_ = plsc.load_gather(table_v, [idx_v[sl]])
        o_v[sl] = jnp.exp(w_v[sl]) * g_
        return _
      lax.fori_loop(0, CH // L, vec_body, None, unroll=4)

    def pair_norm(gp, _):
      g0 = 2 * gp

      @pl.when(gp > 0)
      def _w0():
        wait_out(0)
      wait_in(0)
      norm_chunk(g0, 0)
      start_out(g0, 0)

      @pl.when(gp + 1 < NPAIR)
      def _n0():
        start_in(g0 + 2, 0)

      @pl.when(gp > 0)
      def _w1():
        wait_out(1)
      wait_in(1)
      norm_chunk(g0 + 1, 1)
      start_out(g0 + 1, 1)

      @pl.when(gp + 1 < NPAIR)
      def _n1():
        start_in(g0 + 3, 1)
      return _
    lax.fori_loop(0, NPAIR, pair_norm, None)
    wait_out(0)
    wait_out(1)

  kern = pl.kernel(
      body,
      out_type=(
          jax.ShapeDtypeStruct((NC * NPAD,), jnp.float32),
          jax.ShapeDtypeStruct((E,), jnp.float32),
          jax.ShapeDtypeStruct((E,), jnp.float32),
      ),
      mesh=mesh,
      scratch_types=[
          pltpu.VMEM((NPAD,), jnp.float32),
          pltpu.VMEM((2 * CH,), jnp.int32),
          pltpu.VMEM((2 * CH,), jnp.float32),
          pltpu.VMEM((2 * CH,), jnp.float32),
          pltpu.VMEM((SLICE,), jnp.float32),
          pltpu.VMEM((HSLICE,), jnp.float32),
          pltpu.VMEM_SHARED((NS * HSLICE,), jnp.float32),
          pltpu.VMEM_SHARED((NPAD,), jnp.float32),
          pltpu.SemaphoreType.DMA,
          pltpu.SemaphoreType.DMA,
          pltpu.SemaphoreType.DMA,
          pltpu.SemaphoreType.DMA,
          pltpu.SemaphoreType.DMA,
          pltpu.SemaphoreType.DMA,
          pltpu.SemaphoreType.DMA,
      ],
      compiler_params=pltpu.CompilerParams(needs_layout_passes=False),
      interpret=interpret,
  )
  return kern, NPAD


def kernel(w0, w1, w2, r1, r2):
  E = int(w1.shape[0])
  N = int(w0.shape[0])
  kern, NPAD = _build(E, N, CH=1600)
  w0p = (jnp.full((NC * NPAD + NPAD,), -88.0, jnp.float32)
         .at[:N].set(w0).at[NC * NPAD:].set(0.0))
  lfw0f, lfw1, lfw2 = kern(w0p, w1, w2,
                           r1.astype(jnp.int32), r2.astype(jnp.int32))
  return lfw0f[:N], lfw1, lfw2
